# baseline (speedup 1.0000x reference)
import functools

import jax
import jax.numpy as jnp
from jax import lax
from jax.experimental import pallas as pl
from jax.experimental.pallas import tpu as pltpu
from jax.experimental.pallas import tpu_sc as plsc

NCORES = 2
NSUB = 16
NWORK = NCORES * NSUB
LANES = 16
SEGS = 6
TLEN = 64
NODES = SEGS * (TLEN - 1)
TPAD = 400
TMAX = 377.99997
CH = 16384
UNROLL = 8


def _sc_lut(x, table, dividing_points):
    n = x.shape[0]
    per_w = n // NWORK
    n_chunks = per_w // CH

    mesh = plsc.VectorSubcoreMesh(
        core_axis_name="c", subcore_axis_name="s",
        num_cores=NCORES, num_subcores=NSUB)

    @functools.partial(
        pl.kernel,
        out_type=jax.ShapeDtypeStruct((n,), jnp.float32),
        mesh=mesh,
        scratch_types=[
            pltpu.VMEM((2 * CH,), jnp.float32),
            pltpu.VMEM((2 * CH,), jnp.float32),
            pltpu.VMEM((SEGS, TLEN), jnp.float32),
            pltpu.VMEM((16,), jnp.float32),
            pltpu.VMEM((2, LANES), jnp.float32),
            pltpu.VMEM((TPAD,), jnp.float32),
            pltpu.VMEM((TPAD - LANES,), jnp.float32),
            pltpu.SemaphoreType.DMA,
            pltpu.SemaphoreType.DMA,
        ],
        compiler_params=pltpu.CompilerParams(needs_layout_passes=False),
    )
    def k(x_hbm, tab_hbm, dp_hbm, out_hbm,
          in_v, out_v, raw_v, dp_v, c_v, tab_v, dy_v, in_sem, out_sem):
        wid = lax.axis_index("s") * NCORES + lax.axis_index("c")
        base = wid * per_w
        pltpu.sync_copy(tab_hbm, raw_v)
        pltpu.sync_copy(dp_hbm, dp_v)

        lane = lax.iota(jnp.int32, LANES)
        lo0 = plsc.load_gather(dp_v, [lane * 0])
        hi0 = plsc.load_gather(dp_v, [lane * 0 + SEGS])
        c_v[0] = float(NODES) / (hi0 - lo0)
        c_v[1] = -lo0 * (float(NODES) / (hi0 - lo0))
        inv = c_v[0]
        off = c_v[1]

        @plsc.parallel_loop(0, TPAD, step=LANES)
        def mk_tab(o):
            i = o + lane
            src = jnp.minimum(i + i // (TLEN - 1), SEGS * TLEN - 1)
            tab_v[pl.ds(o, LANES)] = plsc.load_gather(
                raw_v, [src >> 6, src & (TLEN - 1)])

        @plsc.parallel_loop(0, TPAD - LANES, step=LANES)
        def mk_dy(o):
            dy_v[pl.ds(o, LANES)] = (
                tab_v[pl.ds(o + 1, LANES)] - tab_v[pl.ds(o, LANES)])

        def in_copy(g, boff):
            return pltpu.make_async_copy(
                x_hbm.at[pl.ds(base + g * CH, CH)],
                in_v.at[pl.ds(boff, CH)], in_sem)

        def out_copy(g, boff):
            return pltpu.make_async_copy(
                out_v.at[pl.ds(boff, CH)],
                out_hbm.at[pl.ds(base + g * CH, CH)], out_sem)

        def compute(boff):
            @plsc.parallel_loop(0, CH, step=LANES, unroll=UNROLL)
            def vec_body(o):
                xv = in_v[pl.ds(boff + o, LANES)]
                t = jnp.minimum(jnp.maximum(xv * inv + off, 0.0), TMAX)
                ti = t.astype(jnp.int32)
                frac = t - ti.astype(jnp.float32)
                y0 = plsc.load_gather(tab_v, [ti])
                dy = plsc.load_gather(dy_v, [ti])
                out_v[pl.ds(boff + o, LANES)] = y0 + dy * frac

        in_copy(0, 0).start()
        in_copy(1, CH).start()
        for g in (0, 1):
            boff = g * CH
            in_copy(g, boff).wait()
            compute(boff)
            out_copy(g, boff).start()
            in_copy(g + 2, boff).start()

        def steady(g, _):
            boff = (g % 2) * CH
            in_copy(g, boff).wait()
            out_copy(g - 2, boff).wait()
            compute(boff)
            out_copy(g, boff).start()
            in_copy(g + 2, boff).start()
            return 0

        lax.fori_loop(2, n_chunks - 2, steady, 0)
        for g in (n_chunks - 2, n_chunks - 1):
            boff = (g % 2) * CH
            in_copy(g, boff).wait()
            out_copy(g - 2, boff).wait()
            compute(boff)
            out_copy(g, boff).start()
        out_copy(n_chunks - 2, 0).wait()
        out_copy(n_chunks - 1, CH).wait()

    return k(x, table, dividing_points)


def kernel(x, table, dividing_points):
    dp_pad = jnp.zeros((16,), jnp.float32).at[:SEGS + 1].set(dividing_points)
    return _sc_lut(x, table, dp_pad)


# speedup vs baseline: 1.0004x; 1.0004x over previous
import functools

import jax
import jax.numpy as jnp
from jax import lax
from jax.experimental import pallas as pl
from jax.experimental.pallas import tpu as pltpu
from jax.experimental.pallas import tpu_sc as plsc

NCORES = 2
NSUB = 16
NWORK = NCORES * NSUB
LANES = 16
SEGS = 6
TLEN = 64
NODES = SEGS * (TLEN - 1)
TPAD = 400
TMAX = 377.99997
CH = 16384
UNROLL = 8


def _sc_lut(x, table, dividing_points):
    n = x.shape[0]
    per_w = n // NWORK
    n_chunks = per_w // CH

    mesh = plsc.VectorSubcoreMesh(
        core_axis_name="c", subcore_axis_name="s",
        num_cores=NCORES, num_subcores=NSUB)

    @functools.partial(
        pl.kernel,
        out_type=jax.ShapeDtypeStruct((n,), jnp.float32),
        mesh=mesh,
        scratch_types=[
            pltpu.VMEM((2 * CH,), jnp.float32),
            pltpu.VMEM((2 * CH,), jnp.float32),
            pltpu.VMEM((SEGS, TLEN), jnp.float32),
            pltpu.VMEM((16,), jnp.float32),
            pltpu.VMEM((TPAD,), jnp.float32),
            pltpu.VMEM((TPAD - LANES,), jnp.float32),
            pltpu.SemaphoreType.DMA,
            pltpu.SemaphoreType.DMA,
        ],
        compiler_params=pltpu.CompilerParams(needs_layout_passes=False),
    )
    def k(x_hbm, tab_hbm, dp_hbm, out_hbm,
          in_v, out_v, raw_v, dp_v, tab_v, dy_v, in_sem, out_sem):
        wid = lax.axis_index("s") * NCORES + lax.axis_index("c")
        base = wid * per_w
        pltpu.sync_copy(tab_hbm, raw_v)
        pltpu.sync_copy(dp_hbm, dp_v)

        lane = lax.iota(jnp.int32, LANES)
        lo0 = plsc.load_gather(dp_v, [lane * 0])
        hi0 = plsc.load_gather(dp_v, [lane * 0 + SEGS])
        inv = float(NODES) / (hi0 - lo0)
        off = -lo0 * inv

        @plsc.parallel_loop(0, TPAD, step=LANES)
        def mk_tab(o):
            i = o + lane
            src = jnp.minimum(i + i // (TLEN - 1), SEGS * TLEN - 1)
            tab_v[pl.ds(o, LANES)] = plsc.load_gather(
                raw_v, [src >> 6, src & (TLEN - 1)])

        @plsc.parallel_loop(0, TPAD - LANES, step=LANES)
        def mk_dy(o):
            dy_v[pl.ds(o, LANES)] = (
                tab_v[pl.ds(o + 1, LANES)] - tab_v[pl.ds(o, LANES)])

        def in_copy(g, boff):
            return pltpu.make_async_copy(
                x_hbm.at[pl.ds(base + g * CH, CH)],
                in_v.at[pl.ds(boff, CH)], in_sem)

        def out_copy(g, boff):
            return pltpu.make_async_copy(
                out_v.at[pl.ds(boff, CH)],
                out_hbm.at[pl.ds(base + g * CH, CH)], out_sem)

        def compute(boff):
            @plsc.parallel_loop(0, CH, step=LANES, unroll=UNROLL,
                                carry=(inv, off))
            def vec_body(o, c):
                c_inv, c_off = c
                xv = in_v[pl.ds(boff + o, LANES)]
                t = jnp.minimum(jnp.maximum(xv * c_inv + c_off, 0.0), TMAX)
                ti = t.astype(jnp.int32)
                frac = t - ti.astype(jnp.float32)
                y0 = plsc.load_gather(tab_v, [ti])
                dy = plsc.load_gather(dy_v, [ti])
                out_v[pl.ds(boff + o, LANES)] = y0 + dy * frac
                return c

        in_copy(0, 0).start()
        in_copy(1, CH).start()
        for g in (0, 1):
            boff = g * CH
            in_copy(g, boff).wait()
            compute(boff)
            out_copy(g, boff).start()
            in_copy(g + 2, boff).start()

        def steady(g, _):
            boff = (g % 2) * CH
            in_copy(g, boff).wait()
            out_copy(g - 2, boff).wait()
            compute(boff)
            out_copy(g, boff).start()
            in_copy(g + 2, boff).start()
            return 0

        lax.fori_loop(2, n_chunks - 2, steady, 0)
        for g in (n_chunks - 2, n_chunks - 1):
            boff = (g % 2) * CH
            in_copy(g, boff).wait()
            out_copy(g - 2, boff).wait()
            compute(boff)
            out_copy(g, boff).start()
        out_copy(n_chunks - 2, 0).wait()
        out_copy(n_chunks - 1, CH).wait()

    return k(x, table, dividing_points)


def kernel(x, table, dividing_points):
    dp_pad = jnp.zeros((16,), jnp.float32).at[:SEGS + 1].set(dividing_points)
    return _sc_lut(x, table, dp_pad)


# speedup vs baseline: 1.2735x; 1.2730x over previous
import functools

import jax
import jax.numpy as jnp
from jax import lax
from jax.experimental import pallas as pl
from jax.experimental.pallas import tpu as pltpu
from jax.experimental.pallas import tpu_sc as plsc

NCORES = 2
NSUB = 16
NWORK = NCORES * NSUB
LANES = 16
SEGS = 6
TLEN = 64
NODES = SEGS * (TLEN - 1)
TPAD = 400
TMAX = 377.99997
CH = 16384
UNROLL = 16


def _sc_lut(x, tab_merged, consts):
    n = x.shape[0]
    per_w = n // NWORK
    n_chunks = per_w // CH

    mesh = plsc.VectorSubcoreMesh(
        core_axis_name="c", subcore_axis_name="s",
        num_cores=NCORES, num_subcores=NSUB)

    @functools.partial(
        pl.kernel,
        out_type=jax.ShapeDtypeStruct((n,), jnp.float32),
        mesh=mesh,
        scratch_types=[
            pltpu.VMEM((TPAD,), jnp.float32),
            pltpu.VMEM((TPAD - LANES,), jnp.float32),
            pltpu.VMEM((2, LANES), jnp.float32),
            pltpu.VMEM((2 * CH,), jnp.float32),
            pltpu.VMEM((2 * CH,), jnp.float32),
            pltpu.SemaphoreType.DMA,
            pltpu.SemaphoreType.DMA,
        ],
        compiler_params=pltpu.CompilerParams(needs_layout_passes=False),
    )
    def k(x_hbm, tab_hbm, consts_hbm, out_hbm,
          tab_v, dy_v, c_v, in_v, out_v, in_sem, out_sem):
        wid = lax.axis_index("s") * NCORES + lax.axis_index("c")
        base = wid * per_w
        pltpu.sync_copy(tab_hbm, tab_v)
        pltpu.sync_copy(consts_hbm, c_v)
        inv = c_v[0]
        off = c_v[1]

        @plsc.parallel_loop(0, TPAD - LANES, step=LANES)
        def mk_dy(o):
            dy_v[pl.ds(o, LANES)] = (
                tab_v[pl.ds(o + 1, LANES)] - tab_v[pl.ds(o, LANES)])

        def in_copy(g, boff):
            return pltpu.make_async_copy(
                x_hbm.at[pl.ds(base + g * CH, CH)],
                in_v.at[pl.ds(boff, CH)], in_sem)

        def out_copy(g, boff):
            return pltpu.make_async_copy(
                out_v.at[pl.ds(boff, CH)],
                out_hbm.at[pl.ds(base + g * CH, CH)], out_sem)

        def compute(boff):
            @plsc.parallel_loop(0, CH, step=LANES, unroll=UNROLL)
            def vec_body(o):
                xv = in_v[pl.ds(boff + o, LANES)]
                t = jnp.minimum(jnp.maximum(xv * inv + off, 0.0), TMAX)
                ti = t.astype(jnp.int32)
                frac = t - ti.astype(jnp.float32)
                y0 = plsc.load_gather(tab_v, [ti])
                dy = plsc.load_gather(dy_v, [ti])
                out_v[pl.ds(boff + o, LANES)] = y0 + dy * frac

        in_copy(0, 0).start()
        in_copy(1, CH).start()
        for g in (0, 1):
            boff = g * CH
            in_copy(g, boff).wait()
            compute(boff)
            out_copy(g, boff).start()
            in_copy(g + 2, boff).start()

        def steady(g, _):
            boff = (g % 2) * CH
            in_copy(g, boff).wait()
            out_copy(g - 2, boff).wait()
            compute(boff)
            out_copy(g, boff).start()
            in_copy(g + 2, boff).start()
            return 0

        lax.fori_loop(2, n_chunks - 2, steady, 0)
        for g in (n_chunks - 2, n_chunks - 1):
            boff = (g % 2) * CH
            in_copy(g, boff).wait()
            out_copy(g - 2, boff).wait()
            compute(boff)
            out_copy(g, boff).start()
        out_copy(n_chunks - 2, 0).wait()
        out_copy(n_chunks - 1, CH).wait()

    return k(x, tab_merged, consts)


def kernel(x, table, dividing_points):
    tab_merged = jnp.concatenate([
        table[:, : TLEN - 1].reshape(-1),
        table[SEGS - 1:, TLEN - 1],
        jnp.zeros((TPAD - NODES - 1,), jnp.float32),
    ])
    lo0 = dividing_points[0]
    hi0 = dividing_points[-1]
    inv = NODES / (hi0 - lo0)
    consts = jnp.stack([
        jnp.full((LANES,), inv, jnp.float32),
        jnp.full((LANES,), -lo0 * inv, jnp.float32),
    ])
    return _sc_lut(x, tab_merged, consts)
